# baseline (speedup 1.0000x reference)
import functools

import numpy as np
import jax
import jax.numpy as jnp
from jax.experimental import pallas as pl

_ROWS = 64
_V = 32768
_SL = 512
_UNROLL = 4
_NSL = _V // _SL
_L = np.float32(1.0 - 16 * 2.0**-24)
_B = np.float32(2.0**22)
_NCAND = 128


def _st_block(x_ref, o_ref):
    x = x_ref[...]
    inf = jnp.float32(np.inf)

    def _sl(a, k):
        return a[:, k * _SL:(k + 1) * _SL]

    def _tree(vals, op):
        while len(vals) > 1:
            vals = [op(vals[i], vals[i + 1]) for i in range(0, len(vals), 2)]
        return vals[0]

    macc = _tree([_sl(x, k) for k in range(_UNROLL)], jnp.maximum)
    for k0 in range(_UNROLL, _NSL, _UNROLL):
        part = _tree([_sl(x, k0 + j) for j in range(_UNROLL)], jnp.maximum)
        macc = jnp.maximum(macc, part)
    m = jnp.max(macc, axis=1, keepdims=True)

    base = jax.lax.broadcasted_iota(
        jnp.int32, (_ROWS, _SL), 1).astype(jnp.float32)
    sacc = None
    iacc = None
    for k0 in range(0, _NSL, _UNROLL):
        us = [jnp.exp(_sl(x, k0 + j) - m) for j in range(_UNROLL)]
        ids = [jnp.where(us[j] >= _L, base + jnp.float32((k0 + j) * _SL + _B),
                         0.0) for j in range(_UNROLL)]
        spart = _tree(us, jnp.add)
        ipart = _tree(ids, jnp.add)
        sacc = spart if sacc is None else sacc + spart
        iacc = ipart if iacc is None else iacc + ipart
    s = jnp.sum(sacc, axis=1, keepdims=True)
    tot = jnp.sum(iacc, axis=1, keepdims=True)

    umax = jnp.exp(jnp.zeros((_ROWS, 1), jnp.float32))
    pmax = umax / s

    nbad = jnp.any(tot >= 2.0 * _B).astype(jnp.int32)

    def _exact(_, carry):
        k = jax.lax.broadcasted_iota(jnp.int32, (_ROWS, _NCAND), 1)
        ucand = jax.lax.bitcast_convert_type(
            jax.lax.bitcast_convert_type(umax, jnp.int32) - k, jnp.float32)
        in_bucket = (ucand / s) == pmax
        c = jnp.min(jnp.where(in_bucket, ucand, inf), axis=1, keepdims=True)
        u = jnp.exp(x - m)
        fiota = jax.lax.broadcasted_iota(
            jnp.int32, (_ROWS, _V), 1).astype(jnp.float32)
        return jnp.min(jnp.where(u >= c, fiota, inf), axis=1, keepdims=True)

    exact = jax.lax.fori_loop(
        0, nbad, _exact, jnp.full((_ROWS, 1), inf, jnp.float32))
    idx = jnp.where(nbad > 0, exact, tot - _B).astype(jnp.int32)

    v = (1.0 - pmax) + pmax
    iota = jax.lax.broadcasted_iota(jnp.int32, (_ROWS, _V), 1)
    o_ref[...] = jnp.where(iota == idx, v, 0.0)


@jax.jit
def kernel(logits):
    b, h, vocab = logits.shape
    rows = b * h
    x = logits.reshape(rows, vocab)
    out = pl.pallas_call(
        _st_block,
        grid=(rows // _ROWS,),
        in_specs=[pl.BlockSpec((_ROWS, vocab), lambda i: (i, 0))],
        out_specs=pl.BlockSpec((_ROWS, vocab), lambda i: (i, 0)),
        out_shape=jax.ShapeDtypeStruct((rows, vocab), jnp.float32),
    )(x)
    return out.reshape(b, h, vocab)


# speedup vs baseline: 1.0019x; 1.0019x over previous
import functools

import numpy as np
import jax
import jax.numpy as jnp
from jax.experimental import pallas as pl

_ROWS = 64
_V = 32768
_SL = 512
_UNROLL = 8
_NSL = _V // _SL
_L = np.float32(1.0 - 16 * 2.0**-24)
_B = np.float32(2.0**22)
_NCAND = 128


def _st_block(x_ref, o_ref):
    x = x_ref[...]
    inf = jnp.float32(np.inf)

    def _sl(a, k):
        return a[:, k * _SL:(k + 1) * _SL]

    def _tree(vals, op):
        while len(vals) > 1:
            vals = [op(vals[i], vals[i + 1]) for i in range(0, len(vals), 2)]
        return vals[0]

    macc = _tree([_sl(x, k) for k in range(_UNROLL)], jnp.maximum)
    for k0 in range(_UNROLL, _NSL, _UNROLL):
        part = _tree([_sl(x, k0 + j) for j in range(_UNROLL)], jnp.maximum)
        macc = jnp.maximum(macc, part)
    m = jnp.max(macc, axis=1, keepdims=True)

    base = jax.lax.broadcasted_iota(
        jnp.int32, (_ROWS, _SL), 1).astype(jnp.float32)
    sacc = None
    iacc = None
    for k0 in range(0, _NSL, _UNROLL):
        us = [jnp.exp(_sl(x, k0 + j) - m) for j in range(_UNROLL)]
        ids = [jnp.where(us[j] >= _L, base + jnp.float32((k0 + j) * _SL + _B),
                         0.0) for j in range(_UNROLL)]
        spart = _tree(us, jnp.add)
        ipart = _tree(ids, jnp.add)
        sacc = spart if sacc is None else sacc + spart
        iacc = ipart if iacc is None else iacc + ipart
    s = jnp.sum(sacc, axis=1, keepdims=True)
    tot = jnp.sum(iacc, axis=1, keepdims=True)

    umax = jnp.exp(jnp.zeros((_ROWS, 1), jnp.float32))
    pmax = umax / s

    nbad = jnp.any(tot >= 2.0 * _B).astype(jnp.int32)

    def _exact(_, carry):
        k = jax.lax.broadcasted_iota(jnp.int32, (_ROWS, _NCAND), 1)
        ucand = jax.lax.bitcast_convert_type(
            jax.lax.bitcast_convert_type(umax, jnp.int32) - k, jnp.float32)
        in_bucket = (ucand / s) == pmax
        c = jnp.min(jnp.where(in_bucket, ucand, inf), axis=1, keepdims=True)
        u = jnp.exp(x - m)
        fiota = jax.lax.broadcasted_iota(
            jnp.int32, (_ROWS, _V), 1).astype(jnp.float32)
        return jnp.min(jnp.where(u >= c, fiota, inf), axis=1, keepdims=True)

    exact = jax.lax.fori_loop(
        0, nbad, _exact, jnp.full((_ROWS, 1), inf, jnp.float32))
    idx = jnp.where(nbad > 0, exact, tot - _B).astype(jnp.int32)

    v = (1.0 - pmax) + pmax
    iota = jax.lax.broadcasted_iota(jnp.int32, (_ROWS, _V), 1)
    o_ref[...] = jnp.where(iota == idx, v, 0.0)


@jax.jit
def kernel(logits):
    b, h, vocab = logits.shape
    rows = b * h
    x = logits.reshape(rows, vocab)
    out = pl.pallas_call(
        _st_block,
        grid=(rows // _ROWS,),
        in_specs=[pl.BlockSpec((_ROWS, vocab), lambda i: (i, 0))],
        out_specs=pl.BlockSpec((_ROWS, vocab), lambda i: (i, 0)),
        out_shape=jax.ShapeDtypeStruct((rows, vocab), jnp.float32),
    )(x)
    return out.reshape(b, h, vocab)


# speedup vs baseline: 1.0027x; 1.0008x over previous
import functools

import numpy as np
import jax
import jax.numpy as jnp
from jax.experimental import pallas as pl

_ROWS = 64
_V = 32768
_SL = 256
_UNROLL = 4
_NSL = _V // _SL
_L = np.float32(1.0 - 16 * 2.0**-24)
_B = np.float32(2.0**22)
_NCAND = 128


def _st_block(x_ref, o_ref):
    x = x_ref[...]
    inf = jnp.float32(np.inf)

    def _sl(a, k):
        return a[:, k * _SL:(k + 1) * _SL]

    def _tree(vals, op):
        while len(vals) > 1:
            vals = [op(vals[i], vals[i + 1]) for i in range(0, len(vals), 2)]
        return vals[0]

    macc = _tree([_sl(x, k) for k in range(_UNROLL)], jnp.maximum)
    for k0 in range(_UNROLL, _NSL, _UNROLL):
        part = _tree([_sl(x, k0 + j) for j in range(_UNROLL)], jnp.maximum)
        macc = jnp.maximum(macc, part)
    m = jnp.max(macc, axis=1, keepdims=True)

    base = jax.lax.broadcasted_iota(
        jnp.int32, (_ROWS, _SL), 1).astype(jnp.float32)
    sacc = None
    iacc = None
    for k0 in range(0, _NSL, _UNROLL):
        us = [jnp.exp(_sl(x, k0 + j) - m) for j in range(_UNROLL)]
        ids = [jnp.where(us[j] >= _L, base + jnp.float32((k0 + j) * _SL + _B),
                         0.0) for j in range(_UNROLL)]
        spart = _tree(us, jnp.add)
        ipart = _tree(ids, jnp.add)
        sacc = spart if sacc is None else sacc + spart
        iacc = ipart if iacc is None else iacc + ipart
    s = jnp.sum(sacc, axis=1, keepdims=True)
    tot = jnp.sum(iacc, axis=1, keepdims=True)

    umax = jnp.exp(jnp.zeros((_ROWS, 1), jnp.float32))
    pmax = umax / s

    nbad = jnp.any(tot >= 2.0 * _B).astype(jnp.int32)

    def _exact(_, carry):
        k = jax.lax.broadcasted_iota(jnp.int32, (_ROWS, _NCAND), 1)
        ucand = jax.lax.bitcast_convert_type(
            jax.lax.bitcast_convert_type(umax, jnp.int32) - k, jnp.float32)
        in_bucket = (ucand / s) == pmax
        c = jnp.min(jnp.where(in_bucket, ucand, inf), axis=1, keepdims=True)
        u = jnp.exp(x - m)
        fiota = jax.lax.broadcasted_iota(
            jnp.int32, (_ROWS, _V), 1).astype(jnp.float32)
        return jnp.min(jnp.where(u >= c, fiota, inf), axis=1, keepdims=True)

    exact = jax.lax.fori_loop(
        0, nbad, _exact, jnp.full((_ROWS, 1), inf, jnp.float32))
    idx = jnp.where(nbad > 0, exact, tot - _B).astype(jnp.int32)

    v = (1.0 - pmax) + pmax
    iota = jax.lax.broadcasted_iota(jnp.int32, (_ROWS, _V), 1)
    o_ref[...] = jnp.where(iota == idx, v, 0.0)


@jax.jit
def kernel(logits):
    b, h, vocab = logits.shape
    rows = b * h
    x = logits.reshape(rows, vocab)
    out = pl.pallas_call(
        _st_block,
        grid=(rows // _ROWS,),
        in_specs=[pl.BlockSpec((_ROWS, vocab), lambda i: (i, 0))],
        out_specs=pl.BlockSpec((_ROWS, vocab), lambda i: (i, 0)),
        out_shape=jax.ShapeDtypeStruct((rows, vocab), jnp.float32),
    )(x)
    return out.reshape(b, h, vocab)
